# baseline (speedup 1.0000x reference)
import functools

import jax
import jax.numpy as jnp
from jax import lax
from jax.experimental import pallas as pl
from jax.experimental.pallas import tpu as pltpu
from jax.experimental.pallas import tpu_sc as plsc

F = 64
CAV = 5
NY = 504
NX = 504
NP = NY * NX
BLK = 8 * NX
NMACRO = CAV * (NY // 8)
NPIL = 40000
NPAD = 40960
PIDLEN = 1271808
PREG = PIDLEN // 16
PCHUNK = PREG // 4
BPILL = NPAD // 16


def _body(b_hbm, y_hbm, x_hbm, feat_hbm, pid_hbm, out_hbm,
          initbuf, b_v, y_v, x_v, ids_v, dest2d, chk_v,
          pid_v, idl, posl, rows_v, slab0, slab1,
          semA, semS, semG, semW0, semW1):
    c = lax.axis_index("c")
    s = lax.axis_index("s")
    wid = s * 2 + c
    iota = lax.iota(jnp.int32, 16)
    neg1 = jnp.full((16,), -1, jnp.int32)
    zeros16f = jnp.zeros((16,), jnp.float32)
    zeros16i = jnp.zeros((16,), jnp.int32)
    ones16i = jnp.full((16,), 1, jnp.int32)

    def mset_init(i, carry):
        initbuf[pl.ds(i * 16, 16)] = neg1
        return carry

    lax.fori_loop(0, PCHUNK // 16, mset_init, 0)
    pbase = c * PIDLEN
    a_copies = [
        pltpu.async_copy(
            initbuf,
            pid_hbm.at[pl.ds(pbase + s * PREG + k * PCHUNK, PCHUNK)], semA)
        for k in range(4)
    ]

    def mset_slab(slab):
        def step(t, carry):
            slab[pl.ds(t * 16, 16)] = zeros16f
            return carry
        lax.fori_loop(0, 8 * BLK // 16, step, 0)

    mset_slab(slab0)
    mset_slab(slab1)

    def mset_lists(i, carry):
        sl = pl.ds(i * 16, 16)
        idl[sl] = zeros16i
        posl[sl] = zeros16i
        return carry

    lax.fori_loop(0, 4096 // 16, mset_lists, 0)

    c0 = s * BPILL
    pltpu.sync_copy(b_hbm.at[pl.ds(c0, BPILL)], b_v)
    pltpu.sync_copy(y_hbm.at[pl.ds(c0, BPILL)], y_v)
    pltpu.sync_copy(x_hbm.at[pl.ds(c0, BPILL)], x_v)

    def bcompute(i, carry):
        sl = pl.ds(i * 16, 16)
        d = pbase + b_v[sl] * NP + y_v[sl] * NX + x_v[sl]
        dest2d[i // 8, pl.ds((i % 8) * 16, 16)] = d
        ids_v[sl] = c0 + i * 16 + iota
        return carry

    lax.fori_loop(0, BPILL // 16, bcompute, 0)

    for cp in a_copies:
        cp.wait()
    plsc.subcore_barrier()

    s_copies = [
        pltpu.async_copy(ids_v.at[pl.ds(j * 128, 128)],
                         pid_hbm.at[dest2d.at[j]], semS)
        for j in range(BPILL // 128)
    ]
    for cp in s_copies:
        cp.wait()

    def verify_once(_carry):
        def gstep(j, acc):
            pltpu.async_copy(pid_hbm.at[dest2d.at[j]], chk_v, semS).wait()

            def cstep(i, a2):
                neq = (chk_v[pl.ds(i * 16, 16)]
                       != ids_v[pl.ds(j * 128 + i * 16, 16)])
                return a2 + jnp.sum(jnp.where(neq, ones16i, zeros16i))

            return lax.fori_loop(0, 8, cstep, acc)

        return lax.fori_loop(0, BPILL // 128, gstep, jnp.int32(0))

    lax.while_loop(lambda m: m > 0, verify_once, jnp.int32(1))
    plsc.subcore_barrier()

    def compress():
        def step(i, n):
            v = pid_v[pl.ds(i * 16, 16)]
            m = v >= 0
            cs = plsc.cumsum(jnp.where(m, ones16i, zeros16i))
            posv = n + cs - 1
            plsc.store_scatter(idl, [posv], v, mask=m)
            plsc.store_scatter(posl, [posv], i * 16 + iota, mask=m)
            return n + jnp.max(cs)

        return lax.fori_loop(0, BLK // 16, step, jnp.int32(0))

    def assemble(slab, oc, n):
        def gstep(g, carry):
            @pl.when(g * 128 < n)
            def _():
                pltpu.async_copy(
                    feat_hbm.at[idl.at[pl.ds(g * 128, 128)]], rows_v, semG
                ).wait()

                def qstep(q, c2):
                    base = g * 128 + q * 16

                    @pl.when(base < n)
                    def _():
                        pv = posl[pl.ds(base, 16)]
                        mk = (base + iota) < n
                        ridx = q * 16 + iota

                        def cstep(ch, c3):
                            col = oc * 8 + ch + zeros16i
                            val = plsc.load_gather(rows_v, [ridx, col])
                            plsc.store_scatter(
                                slab, [(ch + zeros16i) * BLK + pv], val,
                                mask=mk)
                            return c3

                        lax.fori_loop(0, 8, cstep, 0)
                    return c2

                lax.fori_loop(0, 8, qstep, 0)
            return carry

        lax.fori_loop(0, 32, gstep, 0)

    def rezero(slab, n):
        def tstep(t, carry):
            base = t * 16

            @pl.when(base < n)
            def _():
                pv = posl[pl.ds(base, 16)]
                mk = (base + iota) < n

                def cstep(ch, c2):
                    plsc.store_scatter(
                        slab, [(ch + zeros16i) * BLK + pv], zeros16f, mask=mk)
                    return c2

                lax.fori_loop(0, 8, cstep, 0)
            return carry

        lax.fori_loop(0, BLK // 16, tstep, 0)

    def fire_write(b, yb, oc, slab, semw):
        def wf(ch, carry):
            off = ((b * F + oc * 8 + ch) * NY + yb * 8) * NX
            pltpu.async_copy(slab.at[pl.ds(ch * BLK, BLK)],
                             out_hbm.at[pl.ds(off, BLK)], semw)
            return carry

        lax.fori_loop(0, 8, wf, 0)

    def wait_write(slab, semw):
        def ww(ch, carry):
            pltpu.make_async_copy(slab.at[pl.ds(0, BLK)],
                                  out_hbm.at[pl.ds(0, BLK)], semw).wait()
            return carry

        lax.fori_loop(0, 8, ww, 0)

    def macro_body(k, carry):
        mu = wid + 32 * k

        @pl.when(mu < NMACRO)
        def _():
            b = mu // (NY // 8)
            yb = mu % (NY // 8)
            pltpu.sync_copy(
                pid_hbm.at[pl.ds(pbase + b * NP + yb * BLK, BLK)], pid_v)
            n = compress()

            def pstep(po, c2):
                @pl.when(po > 0)
                def _():
                    wait_write(slab0, semW0)
                    rezero(slab0, n)
                    wait_write(slab1, semW1)
                    rezero(slab1, n)

                assemble(slab0, 2 * po, n)
                fire_write(b, yb, 2 * po, slab0, semW0)
                assemble(slab1, 2 * po + 1, n)
                fire_write(b, yb, 2 * po + 1, slab1, semW1)
                return c2

            lax.fori_loop(0, 4, pstep, 0)
            wait_write(slab0, semW0)
            rezero(slab0, n)
            wait_write(slab1, semW1)
            rezero(slab1, n)
        return carry

    lax.fori_loop(0, 10, macro_body, 0)


@jax.jit
def _run(b_col, y_col, x_col, feat_pad):
    mesh = plsc.VectorSubcoreMesh(core_axis_name="c", subcore_axis_name="s")
    k = functools.partial(
        pl.kernel,
        mesh=mesh,
        compiler_params=pltpu.CompilerParams(needs_layout_passes=False),
        out_type=(
            jax.ShapeDtypeStruct((2 * PIDLEN,), jnp.int32),
            jax.ShapeDtypeStruct((CAV * F * NP,), jnp.float32),
        ),
        scratch_types=[
            pltpu.VMEM((PCHUNK,), jnp.int32),
            pltpu.VMEM((BPILL,), jnp.int32),
            pltpu.VMEM((BPILL,), jnp.int32),
            pltpu.VMEM((BPILL,), jnp.int32),
            pltpu.VMEM((BPILL,), jnp.int32),
            pltpu.VMEM((BPILL // 128, 128), jnp.int32),
            pltpu.VMEM((128,), jnp.int32),
            pltpu.VMEM((BLK,), jnp.int32),
            pltpu.VMEM((4096,), jnp.int32),
            pltpu.VMEM((4096,), jnp.int32),
            pltpu.VMEM((128, 128), jnp.float32),
            pltpu.VMEM((8 * BLK,), jnp.float32),
            pltpu.VMEM((8 * BLK,), jnp.float32),
            pltpu.SemaphoreType.DMA,
            pltpu.SemaphoreType.DMA,
            pltpu.SemaphoreType.DMA,
            pltpu.SemaphoreType.DMA,
            pltpu.SemaphoreType.DMA,
        ],
    )(_body)
    _, out3 = k(b_col, y_col, x_col, feat_pad)
    return out3.reshape(CAV, F, NY, NX)


def kernel(voxel_coords, pillar_features):
    npad = NPAD - NPIL
    ar = jnp.arange(npad, dtype=jnp.int32)
    b_col = jnp.concatenate(
        [voxel_coords[:, 0], jnp.full((npad,), CAV, jnp.int32)])
    y_col = jnp.concatenate([voxel_coords[:, 2], ar // NX])
    x_col = jnp.concatenate([voxel_coords[:, 3], ar % NX])
    feat_pad = jnp.concatenate(
        [pillar_features, jnp.zeros((npad, F), jnp.float32)], axis=0)
    feat_pad = jnp.concatenate(
        [feat_pad, jnp.zeros((NPAD, 128 - F), jnp.float32)], axis=1)
    return _run(b_col.astype(jnp.int32), y_col.astype(jnp.int32),
                x_col.astype(jnp.int32), feat_pad)
